# baseline (speedup 1.0000x reference)
import numpy as np
import jax
import jax.numpy as jnp
from jax import lax
from jax.experimental import pallas as pl
from jax.experimental.pallas import tpu as pltpu
from jax.experimental.pallas import tpu_sc as plsc

N_LEVELS = 16
F_FEAT = 2
LOG2_T = 19
T = 1 << LOG2_T
MASK = T - 1
BASE_RES = 16
PER_LEVEL_SCALE = 1.447
N_NEURONS = 64
N_POINTS = 262144
ENC_DIM = N_LEVELS * F_FEAT

NC = 2
NS = 16
NW = NC * NS
P_PER_W = N_POINTS // NW
CP = 16
NCHUNK = P_PER_W // CP
CROWS = N_LEVELS * 8 * CP

RES = [int(np.floor(BASE_RES * PER_LEVEL_SCALE ** l)) for l in range(N_LEVELS)]
PRIME1 = int(np.uint32(2654435761).view(np.int32))
PRIME2 = int(np.uint32(805459861).view(np.int32))

DEPTH = 4
N_CACHED = 2
HLEV = N_LEVELS - N_CACHED
HROWS = HLEV * 8 * CP
CIDX = N_CACHED * 8 * CP

R1S = [RES[l] + 1 for l in range(N_CACHED)]
DBASE = [0]
for _l in range(N_CACHED):
    DBASE.append(DBASE[-1] + R1S[_l] ** 3)
DREAL = DBASE[-1]
DPAD = 2048 * ((DREAL + 2047) // 2048)


def _dense_idx_const() -> np.ndarray:
    out = np.zeros((DPAD,), dtype=np.uint32)
    for l in range(N_CACHED):
        r1 = R1S[l]
        zz, yy, xx = np.meshgrid(
            np.arange(r1, dtype=np.uint32),
            np.arange(r1, dtype=np.uint32),
            np.arange(r1, dtype=np.uint32),
            indexing="ij",
        )
        h = (xx ^ (yy * np.uint32(2654435761)) ^ (zz * np.uint32(805459861)))
        out[DBASE[l]:DBASE[l + 1]] = (h.reshape(-1) & np.uint32(MASK)) + l * T
    return out.view(np.int32)


_DENSE_IDX = _dense_idx_const()


def _enc_body(xT, tab, didx_hbm, out, xbuf, dcache, didx_v,
              idx0, idx1, idx2, idx3, cdx0, cdx1, cdx2, cdx3,
              w0, w1, w2, w3, row0, row1, row2, row3, encbuf, gsem):
    wid = lax.axis_index("s") * NC + lax.axis_index("c")
    base = pl.multiple_of(wid * P_PER_W, P_PER_W)

    pltpu.sync_copy(didx_hbm, didx_v)
    pltpu.async_copy(tab.at[didx_v], dcache.at[0], gsem)
    pltpu.sync_copy(xT.at[:, pl.ds(base, P_PER_W)], xbuf)
    pltpu.make_async_copy(tab.at[didx_v], dcache.at[0], gsem).wait()

    idxs = [idx0, idx1, idx2, idx3]
    cdxs = [cdx0, cdx1, cdx2, cdx3]
    ws = [w0, w1, w2, w3]
    rows = [row0, row1, row2, row3]

    def compute_and_fire(g, j):
        idxb, cdxb, wb, rowb = idxs[j], cdxs[j], ws[j], rows[j]
        off = g * CP
        px = xbuf[0, pl.ds(off, CP)]
        py = xbuf[1, pl.ds(off, CP)]
        pz = xbuf[2, pl.ds(off, CP)]
        for l in range(N_LEVELS):
            r = float(RES[l])
            posx = px * r
            posy = py * r
            posz = pz * r
            ix = posx.astype(jnp.int32)
            iy = posy.astype(jnp.int32)
            iz = posz.astype(jnp.int32)
            fx = posx - ix.astype(jnp.float32)
            fy = posy - iy.astype(jnp.float32)
            fz = posz - iz.astype(jnp.float32)
            wx = [1.0 - fx, fx]
            wy = [1.0 - fy, fy]
            wz = [1.0 - fz, fz]
            if l < N_CACHED:
                r1 = R1S[l]
                iv = (iz * r1 + iy) * r1 + ix
                for c in range(8):
                    bx, by, bz = c & 1, (c >> 1) & 1, (c >> 2) & 1
                    offd = DBASE[l] + bx + by * r1 + bz * r1 * r1
                    cdxb[pl.ds(l * 128 + 16 * c, 16)] = iv + offd
                    wb[pl.ds(l * 128 + c * 16, 16)] = wx[bx] * wy[by] * wz[bz]
            else:
                hy0 = iy * PRIME1
                hz0 = iz * PRIME2
                hx = [ix, ix + 1]
                hy = [hy0, hy0 + PRIME1]
                hz = [hz0, hz0 + PRIME2]
                hbase = (l - N_CACHED) * 128
                for c in range(8):
                    bx, by, bz = c & 1, (c >> 1) & 1, (c >> 2) & 1
                    idx = ((hx[bx] ^ hy[by] ^ hz[bz]) & MASK) + l * T
                    idxb[pl.ds(hbase + 16 * c, 16)] = idx
                    wb[pl.ds(l * 128 + c * 16, 16)] = wx[bx] * wy[by] * wz[bz]
        pltpu.async_copy(tab.at[idxb], rowb.at[0], gsem)

    zero16 = jnp.zeros((CP,), jnp.int32)

    def process(g, j):
        idxb, cdxb, wb, rowb = idxs[j], cdxs[j], ws[j], rows[j]
        pltpu.make_async_copy(tab.at[idxb], rowb.at[0], gsem).wait()
        sub = g & 7
        col = sub * CP
        for l in range(N_LEVELS):
            acc0 = jnp.zeros((CP,), jnp.float32)
            acc1 = jnp.zeros((CP,), jnp.float32)
            for c in range(8):
                pos = l * 128 + c * 16
                w = wb[pl.ds(pos, 16)]
                if l < N_CACHED:
                    iv = cdxb[pl.ds(pos, 16)]
                    rw = plsc.load_gather(dcache, [zero16, iv])
                else:
                    rw = rowb[0, pl.ds((l - N_CACHED) * 128 + c * 16, 16)]
                acc0 = acc0 + w * plsc.bitcast(rw << 16, jnp.float32)
                acc1 = acc1 + w * plsc.bitcast(rw, jnp.float32)
            encbuf[2 * l, pl.ds(col, CP)] = acc0
            encbuf[2 * l + 1, pl.ds(col, CP)] = acc1

        @pl.when(sub == 7)
        def _flush():
            outoff = pl.multiple_of(base + (g - 7) * CP, 128)
            pltpu.sync_copy(encbuf, out.at[:, pl.ds(outoff, 128)])

    for j in range(DEPTH - 1):
        compute_and_fire(j, j)

    def step(t, carry):
        g0 = t * DEPTH
        for j in range(DEPTH):
            g = g0 + j
            fj = (j + DEPTH - 1) % DEPTH

            @pl.when(g0 + j + DEPTH - 1 < NCHUNK)
            def _fire():
                compute_and_fire(g + DEPTH - 1, fj)

            process(g, j)
        return carry

    lax.fori_loop(0, NCHUNK // DEPTH, step, 0)


_enc_call = pl.kernel(
    _enc_body,
    out_type=jax.ShapeDtypeStruct((ENC_DIM, N_POINTS), jnp.float32),
    mesh=plsc.VectorSubcoreMesh(
        core_axis_name="c", subcore_axis_name="s", num_cores=NC, num_subcores=NS
    ),
    compiler_params=pltpu.CompilerParams(needs_layout_passes=False),
    scratch_types=[
        pltpu.VMEM((3, P_PER_W), jnp.float32),
        pltpu.VMEM((1, DPAD), jnp.int32),
        pltpu.VMEM((DPAD,), jnp.int32),
        pltpu.VMEM((HROWS,), jnp.int32),
        pltpu.VMEM((HROWS,), jnp.int32),
        pltpu.VMEM((HROWS,), jnp.int32),
        pltpu.VMEM((HROWS,), jnp.int32),
        pltpu.VMEM((CIDX,), jnp.int32),
        pltpu.VMEM((CIDX,), jnp.int32),
        pltpu.VMEM((CIDX,), jnp.int32),
        pltpu.VMEM((CIDX,), jnp.int32),
        pltpu.VMEM((CROWS,), jnp.float32),
        pltpu.VMEM((CROWS,), jnp.float32),
        pltpu.VMEM((CROWS,), jnp.float32),
        pltpu.VMEM((CROWS,), jnp.float32),
        pltpu.VMEM((1, HROWS), jnp.int32),
        pltpu.VMEM((1, HROWS), jnp.int32),
        pltpu.VMEM((1, HROWS), jnp.int32),
        pltpu.VMEM((1, HROWS), jnp.int32),
        pltpu.VMEM((ENC_DIM, 128), jnp.float32),
        pltpu.SemaphoreType.DMA,
    ],
)


PB = 2048


def _mlp_body(xT_ref, eT_ref, w0_ref, w1_ref, w2_ref, o_ref):
    xbt = xT_ref[...]
    ebt = eT_ref[...]
    w0 = w0_ref[...]
    dn = (((0,), (0,)), ((), ()))
    h = lax.dot_general(xbt, w0[:3], dn, preferred_element_type=jnp.float32)
    h = h + lax.dot_general(ebt, w0[3:], dn, preferred_element_type=jnp.float32)
    h = jnp.maximum(h, 0.0)
    h = jnp.maximum(jnp.dot(h, w1_ref[...], preferred_element_type=jnp.float32), 0.0)
    o_ref[...] = jnp.dot(h, w2_ref[...], preferred_element_type=jnp.float32) * 0.2


_mlp_call = pl.pallas_call(
    _mlp_body,
    grid=(N_POINTS // PB,),
    in_specs=[
        pl.BlockSpec((3, PB), lambda i: (0, i)),
        pl.BlockSpec((ENC_DIM, PB), lambda i: (0, i)),
        pl.BlockSpec((3 + ENC_DIM, N_NEURONS), lambda i: (0, 0)),
        pl.BlockSpec((N_NEURONS, N_NEURONS), lambda i: (0, 0)),
        pl.BlockSpec((N_NEURONS, 2), lambda i: (0, 0)),
    ],
    out_specs=pl.BlockSpec((PB, 2), lambda i: (i, 0)),
    out_shape=jax.ShapeDtypeStruct((N_POINTS, 2), jnp.float32),
)


def kernel(x, table, W0, W1, W2):
    xT = x.T
    tab_packed = jax.lax.bitcast_convert_type(
        table.astype(jnp.bfloat16).reshape(N_LEVELS * T, F_FEAT), jnp.int32
    )
    didx = jnp.asarray(_DENSE_IDX)
    encT = _enc_call(xT, tab_packed, didx)
    return _mlp_call(xT, encT, W0, W1, W2)


# speedup vs baseline: 1.1652x; 1.1652x over previous
import numpy as np
import jax
import jax.numpy as jnp
from jax import lax
from jax.experimental import pallas as pl
from jax.experimental.pallas import tpu as pltpu
from jax.experimental.pallas import tpu_sc as plsc

N_LEVELS = 16
F_FEAT = 2
LOG2_T = 19
T = 1 << LOG2_T
MASK = T - 1
BASE_RES = 16
PER_LEVEL_SCALE = 1.447
N_NEURONS = 64
N_POINTS = 262144
ENC_DIM = N_LEVELS * F_FEAT

NC = 2
NS = 16
NW = NC * NS
P_PER_W = N_POINTS // NW
CP = 16
NCHUNK = P_PER_W // CP
CROWS = N_LEVELS * 8 * CP

RES = [int(np.floor(BASE_RES * PER_LEVEL_SCALE ** l)) for l in range(N_LEVELS)]
PRIME1 = int(np.uint32(2654435761).view(np.int32))
PRIME2 = int(np.uint32(805459861).view(np.int32))

DEPTH = 8


def _enc_body(xT, tab, out, xbuf, *bufs):
    encbuf = bufs[3 * DEPTH]
    gsem = bufs[3 * DEPTH + 1]
    idxs = bufs[0:DEPTH]
    ws = bufs[DEPTH:2 * DEPTH]
    rows = bufs[2 * DEPTH:3 * DEPTH]

    wid = lax.axis_index("s") * NC + lax.axis_index("c")
    base = pl.multiple_of(wid * P_PER_W, P_PER_W)
    pltpu.sync_copy(xT.at[:, pl.ds(base, P_PER_W)], xbuf)

    def compute_and_fire(g, j):
        idxb, wb, rowb = idxs[j], ws[j], rows[j]
        off = g * CP
        px = xbuf[0, pl.ds(off, CP)]
        py = xbuf[1, pl.ds(off, CP)]
        pz = xbuf[2, pl.ds(off, CP)]
        for l in range(N_LEVELS):
            r = float(RES[l])
            posx = px * r
            posy = py * r
            posz = pz * r
            ix = posx.astype(jnp.int32)
            iy = posy.astype(jnp.int32)
            iz = posz.astype(jnp.int32)
            fx = posx - ix.astype(jnp.float32)
            fy = posy - iy.astype(jnp.float32)
            fz = posz - iz.astype(jnp.float32)
            hy0 = iy * PRIME1
            hz0 = iz * PRIME2
            hx = [ix, ix + 1]
            hy = [hy0, hy0 + PRIME1]
            hz = [hz0, hz0 + PRIME2]
            wx = [1.0 - fx, fx]
            wy = [1.0 - fy, fy]
            wz = [1.0 - fz, fz]
            for c in range(8):
                bx, by, bz = c & 1, (c >> 1) & 1, (c >> 2) & 1
                idx = ((hx[bx] ^ hy[by] ^ hz[bz]) & MASK) + l * T
                idxb[pl.ds(l * 128 + 16 * c, 16)] = idx
                wb[pl.ds(l * 128 + c * 16, 16)] = wx[bx] * wy[by] * wz[bz]
        pltpu.async_copy(tab.at[idxb], rowb.at[0], gsem)

    def process(g, j):
        idxb, wb, rowb = idxs[j], ws[j], rows[j]
        pltpu.make_async_copy(tab.at[idxb], rowb.at[0], gsem).wait()
        sub = g & 7
        col = sub * CP
        for l in range(N_LEVELS):
            acc0 = jnp.zeros((CP,), jnp.float32)
            acc1 = jnp.zeros((CP,), jnp.float32)
            for c in range(8):
                pos = l * 128 + c * 16
                w = wb[pl.ds(pos, 16)]
                rw = rowb[0, pl.ds(pos, 16)]
                acc0 = acc0 + w * plsc.bitcast(rw << 16, jnp.float32)
                acc1 = acc1 + w * plsc.bitcast(rw, jnp.float32)
            encbuf[2 * l, pl.ds(col, CP)] = acc0
            encbuf[2 * l + 1, pl.ds(col, CP)] = acc1

        @pl.when(sub == 7)
        def _flush():
            outoff = pl.multiple_of(base + (g - 7) * CP, 128)
            pltpu.sync_copy(encbuf, out.at[:, pl.ds(outoff, 128)])

    for j in range(DEPTH - 1):
        compute_and_fire(j, j)

    def step(t, carry):
        g0 = t * DEPTH
        for j in range(DEPTH):
            g = g0 + j
            fj = (j + DEPTH - 1) % DEPTH

            @pl.when(g0 + j + DEPTH - 1 < NCHUNK)
            def _fire():
                compute_and_fire(g + DEPTH - 1, fj)

            process(g, j)
        return carry

    lax.fori_loop(0, NCHUNK // DEPTH, step, 0)


_enc_call = pl.kernel(
    _enc_body,
    out_type=jax.ShapeDtypeStruct((ENC_DIM, N_POINTS), jnp.float32),
    mesh=plsc.VectorSubcoreMesh(
        core_axis_name="c", subcore_axis_name="s", num_cores=NC, num_subcores=NS
    ),
    compiler_params=pltpu.CompilerParams(needs_layout_passes=False),
    scratch_types=(
        [pltpu.VMEM((3, P_PER_W), jnp.float32)]
        + [pltpu.VMEM((CROWS,), jnp.int32) for _ in range(DEPTH)]
        + [pltpu.VMEM((CROWS,), jnp.float32) for _ in range(DEPTH)]
        + [pltpu.VMEM((1, CROWS), jnp.int32) for _ in range(DEPTH)]
        + [pltpu.VMEM((ENC_DIM, 128), jnp.float32), pltpu.SemaphoreType.DMA]
    ),
)


PB = 2048


def _mlp_body(xT_ref, eT_ref, w0_ref, w1_ref, w2_ref, o_ref):
    xbt = xT_ref[...]
    ebt = eT_ref[...]
    w0 = w0_ref[...]
    dn = (((0,), (0,)), ((), ()))
    h = lax.dot_general(xbt, w0[:3], dn, preferred_element_type=jnp.float32)
    h = h + lax.dot_general(ebt, w0[3:], dn, preferred_element_type=jnp.float32)
    h = jnp.maximum(h, 0.0)
    h = jnp.maximum(jnp.dot(h, w1_ref[...], preferred_element_type=jnp.float32), 0.0)
    o_ref[...] = jnp.dot(h, w2_ref[...], preferred_element_type=jnp.float32) * 0.2


_mlp_call = pl.pallas_call(
    _mlp_body,
    grid=(N_POINTS // PB,),
    in_specs=[
        pl.BlockSpec((3, PB), lambda i: (0, i)),
        pl.BlockSpec((ENC_DIM, PB), lambda i: (0, i)),
        pl.BlockSpec((3 + ENC_DIM, N_NEURONS), lambda i: (0, 0)),
        pl.BlockSpec((N_NEURONS, N_NEURONS), lambda i: (0, 0)),
        pl.BlockSpec((N_NEURONS, 2), lambda i: (0, 0)),
    ],
    out_specs=pl.BlockSpec((PB, 2), lambda i: (i, 0)),
    out_shape=jax.ShapeDtypeStruct((N_POINTS, 2), jnp.float32),
)


def kernel(x, table, W0, W1, W2):
    xT = x.T
    tab_packed = jax.lax.bitcast_convert_type(
        table.astype(jnp.bfloat16).reshape(N_LEVELS * T, F_FEAT), jnp.int32
    )
    encT = _enc_call(xT, tab_packed)
    return _mlp_call(xT, encT, W0, W1, W2)
